# baseline (speedup 1.0000x reference)
import functools

import jax
import jax.numpy as jnp
from jax import lax
from jax.experimental import pallas as pl
from jax.experimental.pallas import tpu as pltpu
from jax.experimental.pallas import tpu_sc as plsc

N = 4096
D = 32
K = 32
SCALE = 20.0
BLK = 256
GRID = N // BLK
BIG = 3.0e38


def _knn_density_body(xb_ref, xall_ref, xcopy_ref, dens_ref, knn_ref):
    xb = xb_ref[...]
    xa = xall_ref[...]
    sq_b = jnp.sum(xb * xb, axis=1, keepdims=True)
    sq_a = jnp.sum(xa * xa, axis=1, keepdims=True)
    ones = jnp.ones((BLK, 1), jnp.float32)
    a_aug = jnp.concatenate([xb, ones], axis=1)
    b_aug = jnp.concatenate([-2.0 * xa, sq_a], axis=1)
    cross = lax.dot_general(a_aug, b_aug, (((1,), (1,)), ((), ())),
                            preferred_element_type=jnp.float32,
                            precision=lax.Precision.HIGHEST)
    d2 = jnp.maximum(cross + sq_b, 0.0)

    iota_j = lax.broadcasted_iota(jnp.int32, (BLK, N), 1)
    dens_acc = jnp.zeros((BLK, 1), jnp.float32)
    cols = []
    for _ in range(K):
        m = jnp.min(d2, axis=1, keepdims=True)
        is_min = d2 == m
        idx = jnp.min(jnp.where(is_min, iota_j, N), axis=1, keepdims=True)
        d2 = jnp.where(iota_j == idx, BIG, d2)
        dens_acc = dens_acc + jnp.exp(-m / SCALE)
        cols.append(idx)

    xcopy_ref[...] = xb
    dens_ref[...] = dens_acc / (K * SCALE)
    knn_ref[...] = jnp.concatenate(cols, axis=1)


def _knn_density(x, interpret=False):
    return pl.pallas_call(
        _knn_density_body,
        grid=(GRID,),
        in_specs=[
            pl.BlockSpec((BLK, D), lambda i: (i, 0)),
            pl.BlockSpec((N, D), lambda i: (0, 0)),
        ],
        out_specs=[
            pl.BlockSpec((BLK, D), lambda i: (i, 0)),
            pl.BlockSpec((BLK, 1), lambda i: (i, 0)),
            pl.BlockSpec((BLK, K), lambda i: (i, 0)),
        ],
        out_shape=[
            jax.ShapeDtypeStruct((N, D), jnp.float32),
            jax.ShapeDtypeStruct((N, 1), jnp.float32),
            jax.ShapeDtypeStruct((N, K), jnp.int32),
        ],
        interpret=interpret,
    )(x, x)


def _rank_body(dcol_ref, drow_ref, rank_ref):
    i = pl.program_id(0)
    di = dcol_ref[...]
    dj = drow_ref[...]
    iota_j = lax.broadcasted_iota(jnp.int32, (BLK, N), 1)
    i_row = i * BLK + lax.broadcasted_iota(jnp.int32, (BLK, 1), 0)
    lt = dj < di
    eq_lt = (dj == di) & (iota_j < i_row)
    rank_ref[...] = jnp.sum((lt | eq_lt).astype(jnp.int32), axis=1,
                            keepdims=True)


def _rank(dens_col, dens_row, interpret=False):
    return pl.pallas_call(
        _rank_body,
        grid=(GRID,),
        in_specs=[
            pl.BlockSpec((BLK, 1), lambda i: (i, 0)),
            pl.BlockSpec((1, N), lambda i: (0, 0)),
        ],
        out_specs=pl.BlockSpec((BLK, 1), lambda i: (i, 0)),
        out_shape=jax.ShapeDtypeStruct((N, 1), jnp.int32),
        interpret=interpret,
    )(dens_col, dens_row)


_NC = 2
_NS = 16
_L = 16
_NW = _NC * _NS
_CH = N // _NW


def _sc_relabel_body(x_hbm, rank_hbm, knn_hbm, xout_hbm, pcs_hbm, rips_hbm,
                     xv, rank_v, myrank_v, knnf_v, rips_v, sem1, sem2):
    wid = lax.axis_index("s") * _NC + lax.axis_index("c")
    base = wid * _CH
    pltpu.sync_copy(x_hbm.at[pl.ds(base, _CH)], xv)
    pltpu.sync_copy(rank_hbm, rank_v)
    pltpu.sync_copy(rank_hbm.at[pl.ds(base, _CH)], myrank_v)
    pltpu.sync_copy(knn_hbm.at[pl.ds(base * K, _CH * K)], knnf_v)

    pltpu.sync_copy(xv, xout_hbm.at[pl.ds(base, _CH)])

    pltpu.async_copy(xv, pcs_hbm.at[myrank_v], sem1).wait()

    for r in range(_CH):
        for c in range(K // _L):
            idx = knnf_v[pl.ds(r * K + c * _L, _L)]
            rips_v[r, pl.ds(c * _L, _L)] = plsc.load_gather(rank_v, [idx])
    pltpu.async_copy(rips_v, rips_hbm.at[myrank_v], sem2).wait()


def _sc_relabel(xcopy, rank_flat, knn_flat):
    mesh = plsc.VectorSubcoreMesh(core_axis_name="c", subcore_axis_name="s")
    f = pl.kernel(
        _sc_relabel_body,
        out_type=[
            jax.ShapeDtypeStruct((N, D), jnp.float32),
            jax.ShapeDtypeStruct((N, D), jnp.float32),
            jax.ShapeDtypeStruct((N, K), jnp.int32),
        ],
        mesh=mesh,
        scratch_types=[
            pltpu.VMEM((_CH, D), jnp.float32),
            pltpu.VMEM((N,), jnp.int32),
            pltpu.VMEM((_CH,), jnp.int32),
            pltpu.VMEM((_CH * K,), jnp.int32),
            pltpu.VMEM((_CH, K), jnp.int32),
            pltpu.SemaphoreType.DMA,
            pltpu.SemaphoreType.DMA,
        ],
        compiler_params=pltpu.CompilerParams(
            use_tc_tiling_on_sc=False, needs_layout_passes=False),
    )
    return f(xcopy, rank_flat, knn_flat)


def kernel(x):
    xcopy, dens, knn = _knn_density(x)
    rank = _rank(dens, dens.reshape(1, N))
    x_out, _pc_sorted, _rips = _sc_relabel(
        xcopy, rank.reshape(N), knn.reshape(N * K))
    return x_out


# speedup vs baseline: 1.7042x; 1.7042x over previous
import functools

import jax
import jax.numpy as jnp
from jax import lax
from jax.experimental import pallas as pl
from jax.experimental.pallas import tpu as pltpu
from jax.experimental.pallas import tpu_sc as plsc

N = 4096
D = 32
K = 32
SCALE = 20.0
BLK = 256
GRID = N // BLK
BIG = 3.0e38


def _knn_density_body(xb_ref, xall_ref, xcopy_ref, dens_ref, knn_ref):
    xb = xb_ref[...]
    xa = xall_ref[...]
    sq_b = jnp.sum(xb * xb, axis=1, keepdims=True)
    sq_a = jnp.sum(xa * xa, axis=1, keepdims=True)
    ones = jnp.ones((BLK, 1), jnp.float32)
    a_aug = jnp.concatenate([xb, ones], axis=1)
    b_aug = jnp.concatenate([-2.0 * xa, sq_a], axis=1)
    cross = lax.dot_general(a_aug, b_aug, (((1,), (1,)), ((), ())),
                            preferred_element_type=jnp.float32,
                            precision=lax.Precision.HIGHEST)
    d2 = jnp.maximum(cross + sq_b, 0.0)

    iota_j = lax.broadcasted_iota(jnp.int32, (BLK, N), 1)
    bits = lax.bitcast_convert_type(d2, jnp.int32)
    key = (bits & ~0xFFF) | iota_j
    kmax = jnp.int32(0x7FFFFFFF)
    cols = []
    for _ in range(K):
        m = jnp.min(key, axis=1, keepdims=True)
        cols.append(m & 0xFFF)
        key = jnp.where(key == m, kmax, key)

    sel = key == kmax
    dens = jnp.sum(jnp.where(sel, jnp.exp(-d2 / SCALE), 0.0), axis=1,
                   keepdims=True)

    xcopy_ref[...] = xb
    dens_ref[...] = dens / (K * SCALE)
    knn_ref[...] = jnp.concatenate(cols, axis=1)


def _knn_density(x, interpret=False):
    return pl.pallas_call(
        _knn_density_body,
        grid=(GRID,),
        in_specs=[
            pl.BlockSpec((BLK, D), lambda i: (i, 0)),
            pl.BlockSpec((N, D), lambda i: (0, 0)),
        ],
        out_specs=[
            pl.BlockSpec((BLK, D), lambda i: (i, 0)),
            pl.BlockSpec((BLK, 1), lambda i: (i, 0)),
            pl.BlockSpec((BLK, K), lambda i: (i, 0)),
        ],
        out_shape=[
            jax.ShapeDtypeStruct((N, D), jnp.float32),
            jax.ShapeDtypeStruct((N, 1), jnp.float32),
            jax.ShapeDtypeStruct((N, K), jnp.int32),
        ],
        interpret=interpret,
    )(x, x)


def _rank_body(dcol_ref, drow_ref, rank_ref):
    i = pl.program_id(0)
    di = dcol_ref[...]
    dj = drow_ref[...]
    iota_j = lax.broadcasted_iota(jnp.int32, (BLK, N), 1)
    i_row = i * BLK + lax.broadcasted_iota(jnp.int32, (BLK, 1), 0)
    lt = dj < di
    eq_lt = (dj == di) & (iota_j < i_row)
    rank_ref[...] = jnp.sum((lt | eq_lt).astype(jnp.int32), axis=1,
                            keepdims=True)


def _rank(dens_col, dens_row, interpret=False):
    return pl.pallas_call(
        _rank_body,
        grid=(GRID,),
        in_specs=[
            pl.BlockSpec((BLK, 1), lambda i: (i, 0)),
            pl.BlockSpec((1, N), lambda i: (0, 0)),
        ],
        out_specs=pl.BlockSpec((BLK, 1), lambda i: (i, 0)),
        out_shape=jax.ShapeDtypeStruct((N, 1), jnp.int32),
        interpret=interpret,
    )(dens_col, dens_row)


_NC = 2
_NS = 16
_L = 16
_NW = _NC * _NS
_CH = N // _NW


def _sc_relabel_body(x_hbm, rank_hbm, knn_hbm, xout_hbm, pcs_hbm, rips_hbm,
                     xv, rank_v, myrank_v, knnf_v, rips_v, sem1, sem2):
    wid = lax.axis_index("s") * _NC + lax.axis_index("c")
    base = wid * _CH
    pltpu.sync_copy(x_hbm.at[pl.ds(base, _CH)], xv)
    pltpu.sync_copy(rank_hbm, rank_v)
    pltpu.sync_copy(rank_hbm.at[pl.ds(base, _CH)], myrank_v)
    pltpu.sync_copy(knn_hbm.at[pl.ds(base * K, _CH * K)], knnf_v)

    pltpu.sync_copy(xv, xout_hbm.at[pl.ds(base, _CH)])

    pltpu.async_copy(xv, pcs_hbm.at[myrank_v], sem1).wait()

    for r in range(_CH):
        for c in range(K // _L):
            idx = knnf_v[pl.ds(r * K + c * _L, _L)]
            rips_v[r, pl.ds(c * _L, _L)] = plsc.load_gather(rank_v, [idx])
    pltpu.async_copy(rips_v, rips_hbm.at[myrank_v], sem2).wait()


def _sc_relabel(xcopy, rank_flat, knn_flat):
    mesh = plsc.VectorSubcoreMesh(core_axis_name="c", subcore_axis_name="s")
    f = pl.kernel(
        _sc_relabel_body,
        out_type=[
            jax.ShapeDtypeStruct((N, D), jnp.float32),
            jax.ShapeDtypeStruct((N, D), jnp.float32),
            jax.ShapeDtypeStruct((N, K), jnp.int32),
        ],
        mesh=mesh,
        scratch_types=[
            pltpu.VMEM((_CH, D), jnp.float32),
            pltpu.VMEM((N,), jnp.int32),
            pltpu.VMEM((_CH,), jnp.int32),
            pltpu.VMEM((_CH * K,), jnp.int32),
            pltpu.VMEM((_CH, K), jnp.int32),
            pltpu.SemaphoreType.DMA,
            pltpu.SemaphoreType.DMA,
        ],
        compiler_params=pltpu.CompilerParams(
            use_tc_tiling_on_sc=False, needs_layout_passes=False),
    )
    return f(xcopy, rank_flat, knn_flat)


def kernel(x):
    xcopy, dens, knn = _knn_density(x)
    rank = _rank(dens, dens.reshape(1, N))
    x_out, _pc_sorted, _rips = _sc_relabel(
        xcopy, rank.reshape(N), knn.reshape(N * K))
    return x_out


# speedup vs baseline: 2.4330x; 1.4276x over previous
import functools

import jax
import jax.numpy as jnp
from jax import lax
from jax.experimental import pallas as pl
from jax.experimental.pallas import tpu as pltpu
from jax.experimental.pallas import tpu_sc as plsc

N = 4096
D = 32
K = 32
SCALE = 20.0
BLK = 256
GRID = N // BLK
BIG = 3.0e38


def _knn_density_body(xb_ref, xall_ref, xcopy_ref, dens_ref, knn_ref):
    xb = xb_ref[...]
    xa = xall_ref[...]
    sq_b = jnp.sum(xb * xb, axis=1, keepdims=True)
    sq_a = jnp.sum(xa * xa, axis=1, keepdims=True)
    ones = jnp.ones((BLK, 1), jnp.float32)
    a_aug = jnp.concatenate([xb, ones], axis=1)
    b_aug = jnp.concatenate([-2.0 * xa, sq_a], axis=1)
    cross = lax.dot_general(a_aug, b_aug, (((1,), (1,)), ((), ())),
                            preferred_element_type=jnp.float32,
                            precision=lax.Precision.HIGHEST)
    d2 = jnp.maximum(cross + sq_b, 0.0)

    iota_j = lax.broadcasted_iota(jnp.int32, (BLK, N), 1)
    bits = lax.bitcast_convert_type(d2 + 1.0, jnp.int32)
    key = lax.bitcast_convert_type((bits & ~0xFFF) | iota_j, jnp.float32)
    inf = jnp.float32(3.0e38)
    m = jnp.min(key, axis=1, keepdims=True)
    cols = [lax.bitcast_convert_type(m, jnp.int32) & 0xFFF]
    for _ in range(K - 1):
        m = jnp.min(jnp.where(key > m, key, inf), axis=1, keepdims=True)
        cols.append(lax.bitcast_convert_type(m, jnp.int32) & 0xFFF)

    dens = jnp.sum(jnp.where(key <= m, jnp.exp(-d2 / SCALE), 0.0), axis=1,
                   keepdims=True)

    xcopy_ref[...] = xb
    dens_ref[...] = dens / (K * SCALE)
    knn_ref[...] = jnp.concatenate(cols, axis=1)


def _knn_density(x, interpret=False):
    return pl.pallas_call(
        _knn_density_body,
        grid=(GRID,),
        in_specs=[
            pl.BlockSpec((BLK, D), lambda i: (i, 0)),
            pl.BlockSpec((N, D), lambda i: (0, 0)),
        ],
        out_specs=[
            pl.BlockSpec((BLK, D), lambda i: (i, 0)),
            pl.BlockSpec((BLK, 1), lambda i: (i, 0)),
            pl.BlockSpec((BLK, K), lambda i: (i, 0)),
        ],
        out_shape=[
            jax.ShapeDtypeStruct((N, D), jnp.float32),
            jax.ShapeDtypeStruct((N, 1), jnp.float32),
            jax.ShapeDtypeStruct((N, K), jnp.int32),
        ],
        interpret=interpret,
    )(x, x)


def _rank_body(dcol_ref, drow_ref, rank_ref):
    i = pl.program_id(0)
    di = dcol_ref[...]
    dj = drow_ref[...]
    iota_j = lax.broadcasted_iota(jnp.int32, (BLK, N), 1)
    i_row = i * BLK + lax.broadcasted_iota(jnp.int32, (BLK, 1), 0)
    lt = dj < di
    eq_lt = (dj == di) & (iota_j < i_row)
    rank_ref[...] = jnp.sum((lt | eq_lt).astype(jnp.int32), axis=1,
                            keepdims=True)


def _rank(dens_col, dens_row, interpret=False):
    return pl.pallas_call(
        _rank_body,
        grid=(GRID,),
        in_specs=[
            pl.BlockSpec((BLK, 1), lambda i: (i, 0)),
            pl.BlockSpec((1, N), lambda i: (0, 0)),
        ],
        out_specs=pl.BlockSpec((BLK, 1), lambda i: (i, 0)),
        out_shape=jax.ShapeDtypeStruct((N, 1), jnp.int32),
        interpret=interpret,
    )(dens_col, dens_row)


_NC = 2
_NS = 16
_L = 16
_NW = _NC * _NS
_CH = N // _NW


def _sc_relabel_body(x_hbm, rank_hbm, knn_hbm, xout_hbm, pcs_hbm, rips_hbm,
                     xv, rank_v, myrank_v, knnf_v, rips_v, sem1, sem2):
    wid = lax.axis_index("s") * _NC + lax.axis_index("c")
    base = wid * _CH
    pltpu.sync_copy(x_hbm.at[pl.ds(base, _CH)], xv)
    pltpu.sync_copy(rank_hbm, rank_v)
    pltpu.sync_copy(rank_hbm.at[pl.ds(base, _CH)], myrank_v)
    pltpu.sync_copy(knn_hbm.at[pl.ds(base * K, _CH * K)], knnf_v)

    pltpu.sync_copy(xv, xout_hbm.at[pl.ds(base, _CH)])

    pltpu.async_copy(xv, pcs_hbm.at[myrank_v], sem1).wait()

    for r in range(_CH):
        for c in range(K // _L):
            idx = knnf_v[pl.ds(r * K + c * _L, _L)]
            rips_v[r, pl.ds(c * _L, _L)] = plsc.load_gather(rank_v, [idx])
    pltpu.async_copy(rips_v, rips_hbm.at[myrank_v], sem2).wait()


def _sc_relabel(xcopy, rank_flat, knn_flat):
    mesh = plsc.VectorSubcoreMesh(core_axis_name="c", subcore_axis_name="s")
    f = pl.kernel(
        _sc_relabel_body,
        out_type=[
            jax.ShapeDtypeStruct((N, D), jnp.float32),
            jax.ShapeDtypeStruct((N, D), jnp.float32),
            jax.ShapeDtypeStruct((N, K), jnp.int32),
        ],
        mesh=mesh,
        scratch_types=[
            pltpu.VMEM((_CH, D), jnp.float32),
            pltpu.VMEM((N,), jnp.int32),
            pltpu.VMEM((_CH,), jnp.int32),
            pltpu.VMEM((_CH * K,), jnp.int32),
            pltpu.VMEM((_CH, K), jnp.int32),
            pltpu.SemaphoreType.DMA,
            pltpu.SemaphoreType.DMA,
        ],
        compiler_params=pltpu.CompilerParams(
            use_tc_tiling_on_sc=False, needs_layout_passes=False),
    )
    return f(xcopy, rank_flat, knn_flat)


def kernel(x):
    xcopy, dens, knn = _knn_density(x)
    rank = _rank(dens, dens.reshape(1, N))
    x_out, _pc_sorted, _rips = _sc_relabel(
        xcopy, rank.reshape(N), knn.reshape(N * K))
    return x_out


# speedup vs baseline: 2.4466x; 1.0056x over previous
import functools

import jax
import jax.numpy as jnp
from jax import lax
from jax.experimental import pallas as pl
from jax.experimental.pallas import tpu as pltpu
from jax.experimental.pallas import tpu_sc as plsc

N = 4096
D = 32
K = 32
SCALE = 20.0
BLK = 512
GRID = N // BLK
BIG = 3.0e38


def _knn_density_body(xb_ref, xall_ref, xcopy_ref, dens_ref, knn_ref):
    xb = xb_ref[...]
    xa = xall_ref[...]
    sq_b = jnp.sum(xb * xb, axis=1, keepdims=True)
    sq_a = jnp.sum(xa * xa, axis=1, keepdims=True)
    ones = jnp.ones((BLK, 1), jnp.float32)
    a_aug = jnp.concatenate([xb, ones], axis=1)
    b_aug = jnp.concatenate([-2.0 * xa, sq_a], axis=1)
    cross = lax.dot_general(a_aug, b_aug, (((1,), (1,)), ((), ())),
                            preferred_element_type=jnp.float32,
                            precision=lax.Precision.HIGHEST)
    d2 = jnp.maximum(cross + sq_b, 0.0)

    iota_j = lax.broadcasted_iota(jnp.int32, (BLK, N), 1)
    bits = lax.bitcast_convert_type(d2 + 1.0, jnp.int32)
    key = lax.bitcast_convert_type((bits & ~0xFFF) | iota_j, jnp.float32)
    inf = jnp.float32(3.0e38)
    m = jnp.min(key, axis=1, keepdims=True)
    cols = [lax.bitcast_convert_type(m, jnp.int32) & 0xFFF]
    for _ in range(K - 1):
        m = jnp.min(jnp.where(key > m, key, inf), axis=1, keepdims=True)
        cols.append(lax.bitcast_convert_type(m, jnp.int32) & 0xFFF)

    dens = jnp.sum(jnp.where(key <= m, jnp.exp(-d2 / SCALE), 0.0), axis=1,
                   keepdims=True)

    xcopy_ref[...] = xb
    dens_ref[...] = dens / (K * SCALE)
    knn_ref[...] = jnp.concatenate(cols, axis=1)


def _knn_density(x, interpret=False):
    return pl.pallas_call(
        _knn_density_body,
        grid=(GRID,),
        in_specs=[
            pl.BlockSpec((BLK, D), lambda i: (i, 0)),
            pl.BlockSpec((N, D), lambda i: (0, 0)),
        ],
        out_specs=[
            pl.BlockSpec((BLK, D), lambda i: (i, 0)),
            pl.BlockSpec((BLK, 1), lambda i: (i, 0)),
            pl.BlockSpec((BLK, K), lambda i: (i, 0)),
        ],
        out_shape=[
            jax.ShapeDtypeStruct((N, D), jnp.float32),
            jax.ShapeDtypeStruct((N, 1), jnp.float32),
            jax.ShapeDtypeStruct((N, K), jnp.int32),
        ],
        interpret=interpret,
    )(x, x)


def _rank_body(dcol_ref, drow_ref, rank_ref):
    i = pl.program_id(0)
    di = dcol_ref[...]
    dj = drow_ref[...]
    iota_j = lax.broadcasted_iota(jnp.int32, (BLK, N), 1)
    i_row = i * BLK + lax.broadcasted_iota(jnp.int32, (BLK, 1), 0)
    lt = dj < di
    eq_lt = (dj == di) & (iota_j < i_row)
    rank_ref[...] = jnp.sum((lt | eq_lt).astype(jnp.int32), axis=1,
                            keepdims=True)


def _rank(dens_col, dens_row, interpret=False):
    return pl.pallas_call(
        _rank_body,
        grid=(GRID,),
        in_specs=[
            pl.BlockSpec((BLK, 1), lambda i: (i, 0)),
            pl.BlockSpec((1, N), lambda i: (0, 0)),
        ],
        out_specs=pl.BlockSpec((BLK, 1), lambda i: (i, 0)),
        out_shape=jax.ShapeDtypeStruct((N, 1), jnp.int32),
        interpret=interpret,
    )(dens_col, dens_row)


_NC = 2
_NS = 16
_L = 16
_NW = _NC * _NS
_CH = N // _NW


def _sc_relabel_body(x_hbm, rank_hbm, knn_hbm, xout_hbm, pcs_hbm, rips_hbm,
                     xv, rank_v, myrank_v, knnf_v, rips_v, sem1, sem2):
    wid = lax.axis_index("s") * _NC + lax.axis_index("c")
    base = wid * _CH
    pltpu.sync_copy(x_hbm.at[pl.ds(base, _CH)], xv)
    pltpu.sync_copy(rank_hbm, rank_v)
    pltpu.sync_copy(rank_hbm.at[pl.ds(base, _CH)], myrank_v)
    pltpu.sync_copy(knn_hbm.at[pl.ds(base * K, _CH * K)], knnf_v)

    pltpu.sync_copy(xv, xout_hbm.at[pl.ds(base, _CH)])

    pltpu.async_copy(xv, pcs_hbm.at[myrank_v], sem1).wait()

    for r in range(_CH):
        for c in range(K // _L):
            idx = knnf_v[pl.ds(r * K + c * _L, _L)]
            rips_v[r, pl.ds(c * _L, _L)] = plsc.load_gather(rank_v, [idx])
    pltpu.async_copy(rips_v, rips_hbm.at[myrank_v], sem2).wait()


def _sc_relabel(xcopy, rank_flat, knn_flat):
    mesh = plsc.VectorSubcoreMesh(core_axis_name="c", subcore_axis_name="s")
    f = pl.kernel(
        _sc_relabel_body,
        out_type=[
            jax.ShapeDtypeStruct((N, D), jnp.float32),
            jax.ShapeDtypeStruct((N, D), jnp.float32),
            jax.ShapeDtypeStruct((N, K), jnp.int32),
        ],
        mesh=mesh,
        scratch_types=[
            pltpu.VMEM((_CH, D), jnp.float32),
            pltpu.VMEM((N,), jnp.int32),
            pltpu.VMEM((_CH,), jnp.int32),
            pltpu.VMEM((_CH * K,), jnp.int32),
            pltpu.VMEM((_CH, K), jnp.int32),
            pltpu.SemaphoreType.DMA,
            pltpu.SemaphoreType.DMA,
        ],
        compiler_params=pltpu.CompilerParams(
            use_tc_tiling_on_sc=False, needs_layout_passes=False),
    )
    return f(xcopy, rank_flat, knn_flat)


def kernel(x):
    xcopy, dens, knn = _knn_density(x)
    rank = _rank(dens, dens.reshape(1, N))
    x_out, _pc_sorted, _rips = _sc_relabel(
        xcopy, rank.reshape(N), knn.reshape(N * K))
    return x_out
